# baseline (speedup 1.0000x reference)
import functools
import math

import numpy as np
import jax
import jax.numpy as jnp
from jax import lax
from jax.experimental import pallas as pl
from jax.experimental.pallas import tpu as pltpu
from jax.experimental.pallas import tpu_sc as plsc

_VSZ = 1000000
_DSZ = 64
_B = 1024
_T = 200
_ROWS = _B * _T
_NW = 32
_PER_W = _ROWS // _NW
_CHUNK = 400
_NCHUNK = _PER_W // _CHUNK
_SCALE = math.sqrt(_DSZ)
_GSUB = 128


def _build_pe2() -> np.ndarray:
    log_timescale_increment = math.log(10000.0) / float(_DSZ)
    inv_timescales = np.exp(
        np.arange(0, _DSZ, 2, dtype=np.float32) * -log_timescale_increment)
    pe = np.zeros((_T, _DSZ), dtype=np.float32)
    position = np.arange(0, _T, dtype=np.float32)[:, None]
    pe[:, 0::2] = np.sin(position * inv_timescales)
    pe[:, 1::2] = np.cos(position * inv_timescales)
    return np.concatenate([pe, pe], axis=0)


_PE2 = _build_pe2()

_mesh = plsc.VectorSubcoreMesh(core_axis_name="c", subcore_axis_name="s")


@functools.partial(
    pl.kernel,
    mesh=_mesh,
    compiler_params=pltpu.CompilerParams(use_tc_tiling_on_sc=False),
    out_type=jax.ShapeDtypeStruct((_ROWS, _DSZ), jnp.float32),
    scratch_types=[
        pltpu.VMEM((_CHUNK,), jnp.int32),
        pltpu.VMEM((_CHUNK, _DSZ), jnp.float32),
        pltpu.VMEM((_CHUNK, _DSZ), jnp.float32),
        pltpu.SemaphoreType.DMA,
    ],
)
def _sc_embed(w_hbm, idx_hbm, pe_hbm, out_hbm, idx_v, rows_v, pe_v, sem):
    wid = lax.axis_index("s") * 2 + lax.axis_index("c")
    pltpu.sync_copy(pe_hbm, pe_v)

    def chunk_body(c, carry):
        base = wid * _PER_W + c * _CHUNK
        pltpu.sync_copy(idx_hbm.at[pl.ds(base, _CHUNK)], idx_v)
        for off in range(0, _CHUNK, _GSUB):
            n = min(_GSUB, _CHUNK - off)
            pltpu.async_copy(
                w_hbm.at[idx_v.at[pl.ds(off, n)]],
                rows_v.at[pl.ds(off, n)],
                sem,
            ).wait()

        def grp_body(g, carry2):
            iv = idx_v[pl.ds(g * 16, 16)]
            s = jnp.where(iv == 0, 0.0, _SCALE).astype(jnp.float32)
            for rl in range(16):
                r = g * 16 + rl
                sr = s[rl]
                for q in range(_DSZ // 16):
                    sl = pl.ds(q * 16, 16)
                    rows_v[r, sl] = rows_v[r, sl] * sr + pe_v[r, sl]
            return carry2

        lax.fori_loop(0, _CHUNK // 16, grp_body, 0)
        pltpu.sync_copy(rows_v, out_hbm.at[pl.ds(base, _CHUNK)])
        return carry

    lax.fori_loop(0, _NCHUNK, chunk_body, 0)


def kernel(x, W):
    B, T = x.shape
    assert (B, T) == (_B, _T) and W.shape == (_VSZ, _DSZ)
    xf = x.reshape(-1).astype(jnp.int32)
    out = _sc_embed(W.astype(jnp.float32), xf, jnp.asarray(_PE2))
    return out.reshape(B, T, _DSZ)
